# baseline (speedup 1.0000x reference)
import functools

import jax
import jax.numpy as jnp
from jax import lax
from jax.experimental import pallas as pl
from jax.experimental.pallas import tpu as pltpu
from jax.experimental.pallas import tpu_sc as plsc

BATCH = 16384
FACTOR = 16
MLP_DIM = 64
LINE = 128
NROWS = 1000000

_NC = 2
_NS = 16
_NW = _NC * _NS
_BPW = BATCH // _NW
_CHUNK = 128
_NCHUNK = _BPW // _CHUNK

_CB = 1792
_NFULL = NROWS // _CB
_TAIL = NROWS - _NFULL * _CB
_GRID = _NFULL + 1
_GROWS = _GRID * (_CB // 8)
_MROWS = _GRID * (_CB // 2)


def _repack_body(gT_hbm, mT_hbm, g_out, m_out, g_scr, m_scr, sem):
    p = pl.program_id(0)
    csl = pl.ds(p * _CB, _CB)
    cg = pltpu.make_async_copy(gT_hbm.at[:, csl], g_scr, sem)
    cm = pltpu.make_async_copy(mT_hbm.at[:, csl], m_scr, sem)
    cg.start()
    cm.start()
    cg.wait()
    cm.wait()
    g = g_scr[...]
    m = m_scr[...]
    g_out[...] = pltpu.einshape("f(rj)->r(jf)", g, j=8)
    m_out[...] = pltpu.einshape("f(rj)->r(jf)", m, j=2)


def _repack(gT, mT):
    return pl.pallas_call(
        _repack_body,
        grid=(_NFULL,),
        in_specs=[
            pl.BlockSpec(memory_space=pl.ANY),
            pl.BlockSpec(memory_space=pl.ANY),
        ],
        out_specs=[
            pl.BlockSpec((_CB // 8, LINE), lambda i: (i, 0)),
            pl.BlockSpec((_CB // 2, LINE), lambda i: (i, 0)),
        ],
        out_shape=[
            jax.ShapeDtypeStruct((_GROWS, LINE), jnp.float32),
            jax.ShapeDtypeStruct((_MROWS, LINE), jnp.float32),
        ],
        scratch_shapes=[
            pltpu.VMEM((FACTOR, _CB), jnp.float32),
            pltpu.VMEM((MLP_DIM, _CB), jnp.float32),
            pltpu.SemaphoreType.DMA,
        ],
    )(gT, mT)


def _sc_gather(u8, i8, u2, i2, t_u16, t_i16, t_u64, t_i64):
    mesh = plsc.VectorSubcoreMesh(core_axis_name="c", subcore_axis_name="s")
    out_line = jax.ShapeDtypeStruct((BATCH, LINE), jnp.float32)

    @functools.partial(
        pl.kernel,
        out_type=[out_line, out_line, out_line, out_line],
        mesh=mesh,
        scratch_types=[
            pltpu.VMEM((_BPW,), jnp.int32),
            pltpu.VMEM((_BPW,), jnp.int32),
            pltpu.VMEM((_BPW,), jnp.int32),
            pltpu.VMEM((_BPW,), jnp.int32),
            pltpu.VMEM((_CHUNK, LINE), jnp.float32),
            pltpu.VMEM((_CHUNK, LINE), jnp.float32),
            pltpu.VMEM((_CHUNK, LINE), jnp.float32),
            pltpu.VMEM((_CHUNK, LINE), jnp.float32),
            pltpu.SemaphoreType.DMA,
        ],
    )
    def k(u8_h, i8_h, u2_h, i2_h, tu16_h, ti16_h, tu64_h, ti64_h,
          ou16_h, oi16_h, ou64_h, oi64_h,
          u8v, i8v, u2v, i2v, bu16, bi16, bu64, bi64, sem):
        wid = lax.axis_index("s") * _NC + lax.axis_index("c")
        base = wid * _BPW
        pltpu.sync_copy(u8_h.at[pl.ds(base, _BPW)], u8v)
        pltpu.sync_copy(i8_h.at[pl.ds(base, _BPW)], i8v)
        pltpu.sync_copy(u2_h.at[pl.ds(base, _BPW)], u2v)
        pltpu.sync_copy(i2_h.at[pl.ds(base, _BPW)], i2v)
        for c in range(_NCHUNK):
            sl = pl.ds(c * _CHUNK, _CHUNK)
            copies = [
                pltpu.async_copy(tu16_h.at[u8v.at[sl]], bu16, sem),
                pltpu.async_copy(ti16_h.at[i8v.at[sl]], bi16, sem),
                pltpu.async_copy(tu64_h.at[u2v.at[sl]], bu64, sem),
                pltpu.async_copy(ti64_h.at[i2v.at[sl]], bi64, sem),
            ]
            for cp in copies:
                cp.wait()
            osl = pl.ds(base + c * _CHUNK, _CHUNK)
            pltpu.sync_copy(bu16, ou16_h.at[osl])
            pltpu.sync_copy(bi16, oi16_h.at[osl])
            pltpu.sync_copy(bu64, ou64_h.at[osl])
            pltpu.sync_copy(bi64, oi64_h.at[osl])

    return k(u8, i8, u2, i2, t_u16, t_i16, t_u64, t_i64)


_BB = 2048


def _tc_body(ru16, ri16, ru64, ri64, ou8, oi8, ou2, oi2,
             otg_u, otg_i, otm_u, otm_i, tgu, tgi, tmu, tmi,
             w0a, w0b, b0, w1, b1, w2, b2, wp, bp, out):
    def tailfix(rows, tails, ot, nt):
        ks = lax.broadcasted_iota(jnp.int32, (1, nt), 1)
        oh = (ot[...] == ks).astype(jnp.float32)
        trow = oh @ tails[...]
        return jnp.where(ot[...] >= 0, trow, rows)

    def pick16(rows, off):
        acc = jnp.zeros((_BB, FACTOR), jnp.float32)
        o = off[...]
        for kk in range(8):
            acc = acc + jnp.where(o == kk, rows[:, kk * FACTOR:(kk + 1) * FACTOR], 0.0)
        return acc

    r16u = tailfix(ru16[...], tgu, otg_u, 8)
    r16i = tailfix(ri16[...], tgi, otg_i, 8)
    r64u = tailfix(ru64[...], tmu, otm_u, 32)
    r64i = tailfix(ri64[...], tmi, otm_i, 32)
    ug = pick16(r16u, ou8)
    ig = pick16(r16i, oi8)
    gmf = ug * ig
    hu = jnp.where(ou2[...] == 0, r64u[:, :MLP_DIM], r64u[:, MLP_DIM:])
    hi = jnp.where(oi2[...] == 0, r64i[:, :MLP_DIM], r64i[:, MLP_DIM:])
    h = hu @ w0a[...] + hi @ w0b[...] + b0[...]
    h = jnp.maximum(h, 0.0)
    h = jnp.maximum(h @ w1[...] + b1[...], 0.0)
    h = jnp.maximum(h @ w2[...] + b2[...], 0.0)
    fused = jnp.concatenate([gmf, h], axis=-1)
    out[...] = jnp.sum(fused * wp[...], axis=-1) + bp[0]


def _tc_dense(ru16, ri16, ru64, ri64, ou8, oi8, ou2, oi2,
              otg_u, otg_i, otm_u, otm_i, tgu, tgi, tmu, tmi,
              W0, b0, W1, b1, W2, b2, Wp, bp):
    grid = (BATCH // _BB,)

    def row_blk(shape):
        return pl.BlockSpec((_BB,) + shape[1:], lambda i: (i,) + (0,) * (len(shape) - 1))

    def full_blk(shape):
        return pl.BlockSpec(shape, lambda i: (0,) * len(shape))

    w0a, w0b = W0[:MLP_DIM], W0[MLP_DIM:]
    b0r, b1r, b2r = b0.reshape(1, -1), b1.reshape(1, -1), b2.reshape(1, -1)
    wpr = Wp.reshape(1, -1)
    in_specs = [
        row_blk((BATCH, LINE)), row_blk((BATCH, LINE)),
        row_blk((BATCH, LINE)), row_blk((BATCH, LINE)),
        row_blk((BATCH, 1)), row_blk((BATCH, 1)),
        row_blk((BATCH, 1)), row_blk((BATCH, 1)),
        row_blk((BATCH, 1)), row_blk((BATCH, 1)),
        row_blk((BATCH, 1)), row_blk((BATCH, 1)),
        full_blk(tgu.shape), full_blk(tgi.shape),
        full_blk(tmu.shape), full_blk(tmi.shape),
        full_blk(w0a.shape), full_blk(w0b.shape), full_blk(b0r.shape),
        full_blk(W1.shape), full_blk(b1r.shape),
        full_blk(W2.shape), full_blk(b2r.shape),
        full_blk(wpr.shape), full_blk(bp.shape),
    ]
    return pl.pallas_call(
        _tc_body,
        grid=grid,
        in_specs=in_specs,
        out_specs=pl.BlockSpec((_BB,), lambda i: (i,)),
        out_shape=jax.ShapeDtypeStruct((BATCH,), jnp.float32),
    )(ru16, ri16, ru64, ri64, ou8, oi8, ou2, oi2,
      otg_u, otg_i, otm_u, otm_i, tgu, tgi, tmu, tmi,
      w0a, w0b, b0r, W1, b1r, W2, b2r, wpr, bp)


_MAIN = _NFULL * _CB


def kernel(user, item, user_emb_gmf, item_emb_gmf, user_emb_mlp, item_emb_mlp,
           W0, b0, W1, b1, W2, b2, Wp, bp):
    u = user.astype(jnp.int32)
    it = item.astype(jnp.int32)
    t_u16, t_u64 = _repack(user_emb_gmf.T, user_emb_mlp.T)
    t_i16, t_i64 = _repack(item_emb_gmf.T, item_emb_mlp.T)
    tgu = user_emb_gmf[_MAIN:].reshape(8, LINE)
    tgi = item_emb_gmf[_MAIN:].reshape(8, LINE)
    tmu = user_emb_mlp[_MAIN:].reshape(32, LINE)
    tmi = item_emb_mlp[_MAIN:].reshape(32, LINE)
    u8, i8 = u >> 3, it >> 3
    u2, i2 = u >> 1, it >> 1
    ru16, ri16, ru64, ri64 = _sc_gather(u8, i8, u2, i2, t_u16, t_i16, t_u64, t_i64)
    ou8 = (u & 7).reshape(-1, 1)
    oi8 = (it & 7).reshape(-1, 1)
    ou2 = (u & 1).reshape(-1, 1)
    oi2 = (it & 1).reshape(-1, 1)
    otg_u = (u8 - _MAIN // 8).reshape(-1, 1)
    otg_i = (i8 - _MAIN // 8).reshape(-1, 1)
    otm_u = (u2 - _MAIN // 2).reshape(-1, 1)
    otm_i = (i2 - _MAIN // 2).reshape(-1, 1)
    return _tc_dense(ru16, ri16, ru64, ri64, ou8, oi8, ou2, oi2,
                     otg_u, otg_i, otm_u, otm_i, tgu, tgi, tmu, tmi,
                     W0, b0, W1, b1, W2, b2, Wp, bp)


# speedup vs baseline: 11.4146x; 11.4146x over previous
import functools

import jax
import jax.numpy as jnp
from jax import lax
from jax.experimental import pallas as pl
from jax.experimental.pallas import tpu as pltpu
from jax.experimental.pallas import tpu_sc as plsc

BATCH = 16384
FACTOR = 16
MLP_DIM = 64

_NC = 2
_NS = 16
_NW = _NC * _NS
_BPW = BATCH // _NW
_CHUNK = 128
_NCHUNK = _BPW // _CHUNK


def _sc_gather(user, item, ug_t, ig_t, um_t, im_t):
    mesh = plsc.VectorSubcoreMesh(core_axis_name="c", subcore_axis_name="s")

    @functools.partial(
        pl.kernel,
        out_type=[
            jax.ShapeDtypeStruct((BATCH, FACTOR), jnp.float32),
            jax.ShapeDtypeStruct((BATCH, FACTOR), jnp.float32),
            jax.ShapeDtypeStruct((BATCH, MLP_DIM), jnp.float32),
            jax.ShapeDtypeStruct((BATCH, MLP_DIM), jnp.float32),
        ],
        mesh=mesh,
        compiler_params=pltpu.CompilerParams(use_tc_tiling_on_sc=False),
        scratch_types=[
            pltpu.VMEM((_BPW,), jnp.int32),
            pltpu.VMEM((_BPW,), jnp.int32),
            pltpu.VMEM((_BPW, FACTOR), jnp.float32),
            pltpu.VMEM((_BPW, FACTOR), jnp.float32),
            pltpu.VMEM((_BPW, MLP_DIM), jnp.float32),
            pltpu.VMEM((_BPW, MLP_DIM), jnp.float32),
            pltpu.SemaphoreType.DMA,
        ],
    )
    def k(user_h, item_h, ug_h, ig_h, um_h, im_h,
          oug_h, oig_h, oum_h, oim_h,
          uidx_v, iidx_v, ugr, igr, umr, imr, sem):
        wid = lax.axis_index("s") * _NC + lax.axis_index("c")
        base = wid * _BPW
        pltpu.sync_copy(user_h.at[pl.ds(base, _BPW)], uidx_v)
        pltpu.sync_copy(item_h.at[pl.ds(base, _BPW)], iidx_v)
        copies = []
        for j in range(_NCHUNK):
            sl = pl.ds(j * _CHUNK, _CHUNK)
            copies.append(pltpu.async_copy(ug_h.at[uidx_v.at[sl]], ugr.at[sl], sem))
            copies.append(pltpu.async_copy(ig_h.at[iidx_v.at[sl]], igr.at[sl], sem))
            copies.append(pltpu.async_copy(um_h.at[uidx_v.at[sl]], umr.at[sl], sem))
            copies.append(pltpu.async_copy(im_h.at[iidx_v.at[sl]], imr.at[sl], sem))
        for c in copies:
            c.wait()
        out_sl = pl.ds(base, _BPW)
        pltpu.sync_copy(ugr, oug_h.at[out_sl])
        pltpu.sync_copy(igr, oig_h.at[out_sl])
        pltpu.sync_copy(umr, oum_h.at[out_sl])
        pltpu.sync_copy(imr, oim_h.at[out_sl])

    return k(user, item, ug_t, ig_t, um_t, im_t)


_BB = 2048


def _tc_body(ug_ref, ig_ref, um_ref, im_ref, w0a_ref, w0b_ref, b0_ref,
             w1_ref, b1_ref, w2_ref, b2_ref, wp_ref, bp_ref, out_ref):
    gmf = ug_ref[...] * ig_ref[...]
    h = um_ref[...] @ w0a_ref[...] + im_ref[...] @ w0b_ref[...] + b0_ref[...]
    h = jnp.maximum(h, 0.0)
    h = jnp.maximum(h @ w1_ref[...] + b1_ref[...], 0.0)
    h = jnp.maximum(h @ w2_ref[...] + b2_ref[...], 0.0)
    fused = jnp.concatenate([gmf, h], axis=-1)
    out_ref[...] = jnp.sum(fused * wp_ref[...], axis=-1) + bp_ref[0]


def _tc_dense(ug, ig, um, im, W0, b0, W1, b1, W2, b2, Wp, bp):
    grid = (BATCH // _BB,)

    def row_blk(shape):
        return pl.BlockSpec((_BB,) + shape[1:], lambda i: (i,) + (0,) * (len(shape) - 1))

    def full_blk(shape):
        return pl.BlockSpec(shape, lambda i: (0,) * len(shape))

    w0a, w0b = W0[:MLP_DIM], W0[MLP_DIM:]
    b0r, b1r, b2r = b0.reshape(1, -1), b1.reshape(1, -1), b2.reshape(1, -1)
    wpr = Wp.reshape(1, -1)
    in_specs = [
        row_blk((BATCH, FACTOR)), row_blk((BATCH, FACTOR)),
        row_blk((BATCH, MLP_DIM)), row_blk((BATCH, MLP_DIM)),
        full_blk(w0a.shape), full_blk(w0b.shape), full_blk(b0r.shape),
        full_blk(W1.shape), full_blk(b1r.shape),
        full_blk(W2.shape), full_blk(b2r.shape),
        full_blk(wpr.shape), full_blk(bp.shape),
    ]
    return pl.pallas_call(
        _tc_body,
        grid=grid,
        in_specs=in_specs,
        out_specs=pl.BlockSpec((_BB,), lambda i: (i,)),
        out_shape=jax.ShapeDtypeStruct((BATCH,), jnp.float32),
    )(ug, ig, um, im, w0a, w0b, b0r, W1, b1r, W2, b2r, wpr, bp)


def kernel(user, item, user_emb_gmf, item_emb_gmf, user_emb_mlp, item_emb_mlp,
           W0, b0, W1, b1, W2, b2, Wp, bp):
    user = user.astype(jnp.int32)
    item = item.astype(jnp.int32)
    ug, ig, um, im = _sc_gather(user, item, user_emb_gmf, item_emb_gmf,
                                user_emb_mlp, item_emb_mlp)
    return _tc_dense(ug, ig, um, im, W0, b0, W1, b1, W2, b2, Wp, bp)
